# baseline (speedup 1.0000x reference)
import functools

import jax
import jax.numpy as jnp
from jax import lax
from jax.experimental import pallas as pl
from jax.experimental.pallas import tpu as pltpu
from jax.experimental.pallas import tpu_sc as plsc

VOCAB = 1000000
EMB = 64
N_TOK = 1024 * 200

_NC = 2
_NS = 16
_NW = _NC * _NS

_PER_W = N_TOK // _NW
_CHUNK = 800
_NCHUNK = _PER_W // _CHUNK

_TBLK = 8192
_Q = 262144
_NTB = _Q // _TBLK
_IN_BLKS = -(-VOCAB // _TBLK)


def _transpose_body(x0_ref, x1_ref, x2_ref, x3_ref, out_ref):
    eye = jnp.eye(EMB, dtype=jnp.bfloat16)

    def tp(ref):
        xb = ref[...].astype(jnp.bfloat16)
        y = lax.dot_general(xb, eye, (((0,), (0,)), ((), ())),
                            preferred_element_type=jnp.float32)
        yb = y.astype(jnp.bfloat16)
        return lax.bitcast_convert_type(yb, jnp.uint16).astype(jnp.uint32)

    a, b, c, d = tp(x0_ref), tp(x1_ref), tp(x2_ref), tp(x3_ref)
    out_ref[...] = jnp.concatenate([a | (b << 16), c | (d << 16)], axis=1)


def _emb_body(src_hbm, table_hbm, out_hbm, idx_v, buf0, buf1, gsem0, gsem1,
              wsem0, wsem1):
    wid = lax.axis_index("s") * _NC + lax.axis_index("c")
    base = wid * _PER_W
    pltpu.sync_copy(src_hbm.at[pl.ds(base, _PER_W)], idx_v)

    bufs = (buf0, buf1)
    gsems = (gsem0, gsem1)
    wsems = (wsem0, wsem1)

    def gather(c):
        b = c % 2
        return pltpu.async_copy(
            table_hbm.at[idx_v.at[pl.ds(c * _CHUNK, _CHUNK)]], bufs[b],
            gsems[b])

    def write(c):
        b = c % 2
        return pltpu.async_copy(
            bufs[b], out_hbm.at[pl.ds(base + c * _CHUNK, _CHUNK)], wsems[b])

    g = [None] * _NCHUNK
    w = [None] * _NCHUNK
    g[0] = gather(0)
    g[1] = gather(1)
    for c in range(_NCHUNK):
        g[c].wait()
        w[c] = write(c)
        if c + 2 < _NCHUNK:
            w[c].wait()
            g[c + 2] = gather(c + 2)
    w[_NCHUNK - 2].wait()
    w[_NCHUNK - 1].wait()


@jax.jit
def _embedding_lookup(src_flat, table):
    def in_spec(k):
        return pl.BlockSpec(
            (EMB, _TBLK),
            lambda i, k=k: (0, jnp.minimum(i + k * _NTB, _IN_BLKS - 1)))

    tt = table.T
    table_pairs = pl.pallas_call(
        _transpose_body,
        grid=(_NTB,),
        in_specs=[in_spec(0), in_spec(1), in_spec(2), in_spec(3)],
        out_specs=pl.BlockSpec((_TBLK, 128), lambda i: (i, 0)),
        out_shape=jax.ShapeDtypeStruct((_Q, 128), jnp.uint32),
    )(tt, tt, tt, tt)
    table_rm = table_pairs.reshape(2 * _Q, EMB)

    mesh = plsc.VectorSubcoreMesh(core_axis_name="c", subcore_axis_name="s")
    fn = functools.partial(
        pl.kernel,
        mesh=mesh,
        out_type=jax.ShapeDtypeStruct((N_TOK, EMB), jnp.uint32),
        scratch_types=[
            pltpu.VMEM((_PER_W,), jnp.int32),
            pltpu.VMEM((_CHUNK, EMB), jnp.uint32),
            pltpu.VMEM((_CHUNK, EMB), jnp.uint32),
            pltpu.SemaphoreType.DMA,
            pltpu.SemaphoreType.DMA,
            pltpu.SemaphoreType.DMA,
            pltpu.SemaphoreType.DMA,
        ],
        compiler_params=pltpu.CompilerParams(use_tc_tiling_on_sc=False),
    )(_emb_body)
    return fn(src_flat, table_rm)


def kernel(src, seg, table):
    del seg
    v = src.reshape(-1).astype(jnp.int32)
    part = v >> 18
    g = part >> 1
    h = (part & 1).astype(jnp.uint32)
    idx = 2 * (v & (_Q - 1)) + g
    raw = _embedding_lookup(idx, table)
    bits = ((raw >> (h[:, None] * 16)) & jnp.uint32(0xFFFF)) << 16
    out = lax.bitcast_convert_type(bits, jnp.float32)
    return out.reshape(src.shape[0], src.shape[1], EMB)


# speedup vs baseline: 1.0080x; 1.0080x over previous
import functools

import jax
import jax.numpy as jnp
from jax import lax
from jax.experimental import pallas as pl
from jax.experimental.pallas import tpu as pltpu
from jax.experimental.pallas import tpu_sc as plsc

VOCAB = 1000000
EMB = 64
N_TOK = 1024 * 200

_NC = 2
_NS = 16
_NW = _NC * _NS

_PER_W = N_TOK // _NW
_CHUNK = 800
_NCHUNK = _PER_W // _CHUNK

_TBLK = 8192
_WIN = 4 * _TBLK
_NTB = -(-VOCAB // _WIN)
_QROWS = _NTB * _TBLK


def _transpose_body(x_ref, out_ref):
    eye = jnp.eye(EMB, dtype=jnp.bfloat16)

    def tp(k):
        xb = x_ref[:, k * _TBLK:(k + 1) * _TBLK].astype(jnp.bfloat16)
        y = lax.dot_general(xb, eye, (((0,), (0,)), ((), ())),
                            preferred_element_type=jnp.float32)
        yb = y.astype(jnp.bfloat16)
        return lax.bitcast_convert_type(yb, jnp.uint16).astype(jnp.uint32)

    a, b, c, d = tp(0), tp(1), tp(2), tp(3)
    out_ref[...] = jnp.concatenate([a | (b << 16), c | (d << 16)], axis=1)


def _emb_body(src_hbm, table_hbm, out_hbm, idx_v, buf0, buf1, gsem0, gsem1,
              wsem0, wsem1):
    wid = lax.axis_index("s") * _NC + lax.axis_index("c")
    base = wid * _PER_W
    pltpu.sync_copy(src_hbm.at[pl.ds(base, _PER_W)], idx_v)

    bufs = (buf0, buf1)
    gsems = (gsem0, gsem1)
    wsems = (wsem0, wsem1)

    def gather(c):
        b = c % 2
        return pltpu.async_copy(
            table_hbm.at[idx_v.at[pl.ds(c * _CHUNK, _CHUNK)]], bufs[b],
            gsems[b])

    def write(c):
        b = c % 2
        return pltpu.async_copy(
            bufs[b], out_hbm.at[pl.ds(base + c * _CHUNK, _CHUNK)], wsems[b])

    g = [None] * _NCHUNK
    w = [None] * _NCHUNK
    g[0] = gather(0)
    g[1] = gather(1)
    for c in range(_NCHUNK):
        g[c].wait()
        w[c] = write(c)
        if c + 2 < _NCHUNK:
            w[c].wait()
            g[c + 2] = gather(c + 2)
    w[_NCHUNK - 2].wait()
    w[_NCHUNK - 1].wait()


@jax.jit
def _embedding_lookup(src_flat, table):
    table_pairs = pl.pallas_call(
        _transpose_body,
        grid=(_NTB,),
        in_specs=[pl.BlockSpec((EMB, _WIN), lambda i: (0, i))],
        out_specs=pl.BlockSpec((_TBLK, 128), lambda i: (i, 0)),
        out_shape=jax.ShapeDtypeStruct((_QROWS, 128), jnp.uint32),
    )(table.T)
    table_rm = table_pairs.reshape(2 * _QROWS, EMB)

    mesh = plsc.VectorSubcoreMesh(core_axis_name="c", subcore_axis_name="s")
    fn = functools.partial(
        pl.kernel,
        mesh=mesh,
        out_type=jax.ShapeDtypeStruct((N_TOK, EMB), jnp.uint32),
        scratch_types=[
            pltpu.VMEM((_PER_W,), jnp.int32),
            pltpu.VMEM((_CHUNK, EMB), jnp.uint32),
            pltpu.VMEM((_CHUNK, EMB), jnp.uint32),
            pltpu.SemaphoreType.DMA,
            pltpu.SemaphoreType.DMA,
            pltpu.SemaphoreType.DMA,
            pltpu.SemaphoreType.DMA,
        ],
        compiler_params=pltpu.CompilerParams(use_tc_tiling_on_sc=False),
    )(_emb_body)
    return fn(src_flat, table_rm)


def kernel(src, seg, table):
    del seg
    v = src.reshape(-1).astype(jnp.int32)
    part = (v >> 13) & 3
    g = part >> 1
    h = (part & 1).astype(jnp.uint32)
    r = ((v >> 15) << 13) | (v & (_TBLK - 1))
    idx = 2 * r + g
    raw = _embedding_lookup(idx, table)
    bits = ((raw >> (h[:, None] * 16)) & jnp.uint32(0xFFFF)) << 16
    out = lax.bitcast_convert_type(bits, jnp.float32)
    return out.reshape(src.shape[0], src.shape[1], EMB)
